# baseline (speedup 1.0000x reference)
import functools

import jax
import jax.numpy as jnp
from jax import lax
from jax.experimental import pallas as pl
from jax.experimental.pallas import tpu as pltpu
from jax.experimental.pallas import tpu_sc as plsc

NC = 2
NS = 16
L = 16
NW = NC * NS
CW = 512
MCAP = 3072
RING = 64


def kernel(target_nodes, context_nodes, embeddings, context_embeddings):
    B = target_nodes.shape[0]
    N, D = embeddings.shape
    NT = (N + 127) // 128
    RMAX0 = ((N + 127) // 128) * 128 - CW
    BPW = B // NW

    mesh = plsc.VectorSubcoreMesh(core_axis_name="c", subcore_axis_name="s")

    @functools.partial(
        pl.kernel,
        mesh=mesh,
        compiler_params=pltpu.CompilerParams(needs_layout_passes=False),
        out_type=jax.ShapeDtypeStruct((B, 2 * D), jnp.float32),
        scratch_types=[
            pltpu.VMEM((B,), jnp.int32),
            pltpu.VMEM((B,), jnp.int32),
            pltpu.VMEM((MCAP,), jnp.int32),
            pltpu.VMEM((MCAP,), jnp.int32),
            pltpu.VMEM((MCAP,), jnp.int32),
            pltpu.VMEM((MCAP,), jnp.int32),
            pltpu.VMEM((2, D, CW), jnp.float32),
            pltpu.VMEM((RING * D,), jnp.float32),
            pltpu.SemaphoreType.DMA,
            pltpu.SemaphoreType.DMA,
            pltpu.SemaphoreType.DMA,
        ],
    )
    def gather_kernel(tgt_hbm, ctx_hbm, embt_hbm, cembt_hbm, stag_hbm,
                      tval_v, cval_v, mr0_v, me0_v, mr1_v, me1_v, win, ring,
                      sem0, sem1, ssem):
        wid = lax.axis_index("s") * NC + lax.axis_index("c")
        t_lo = (wid * NT) // NW
        t_hi = ((wid + 1) * NT) // NW
        lo_abs = t_lo * 128
        hi_abs = jnp.minimum(t_hi * 128, N)
        nch = (hi_abs - lo_abs + CW - 1) // CW

        iot = lax.iota(jnp.int32, L)

        def start_win(tab_hbm, cc, p):
            r0c = pl.multiple_of(
                jnp.minimum(lo_abs + cc * CW, RMAX0), 128)
            for q in range(4):
                src = tab_hbm.at[pl.ds(0, D), pl.ds(r0c + q * 128, 128)]

                @pl.when(p == 0)
                def _():
                    pltpu.async_copy(src, win.at[0, pl.ds(0, D),
                                                 pl.ds(q * 128, 128)], sem0)

                @pl.when(p == 1)
                def _():
                    pltpu.async_copy(src, win.at[1, pl.ds(0, D),
                                                 pl.ds(q * 128, 128)], sem1)

        start_win(embt_hbm, 0, 0)

        pltpu.sync_copy(tgt_hbm, tval_v)
        pltpu.sync_copy(ctx_hbm, cval_v)

        def mk_scan(val_v, mr_v, me_v):
            def scan_body(g, cur):
                v = val_v[pl.ds(g * L, L)]
                pos = g * L + iot
                m = (v >= lo_abs) & (v < hi_abs)
                plsc.store_compressed(mr_v.at[pl.ds(cur, L)], v, mask=m)
                plsc.store_compressed(me_v.at[pl.ds(cur, L)], pos, mask=m)
                return cur + plsc.all_reduce_population_count(m)[0]
            return scan_body

        cnt_t = lax.fori_loop(0, B // L, mk_scan(tval_v, mr0_v, me0_v), 0)
        cnt_c = lax.fori_loop(0, B // L, mk_scan(cval_v, mr1_v, me1_v), 0)

        def run_pass(tab_hbm, mr_v, me_v, cnt, colbase, mc0):
            mg = (cnt + L - 1) // L

            def chunk_body(cc, mc):
                p = cc % 2
                chunk_lo = lo_abs + cc * CW
                chunk_hi = jnp.minimum(chunk_lo + CW, hi_abs)
                r0c = jnp.minimum(chunk_lo, RMAX0)

                @pl.when(cc + 1 < nch)
                def _():
                    start_win(tab_hbm, cc + 1, 1 - p)

                @pl.when(p == 0)
                def _():
                    pltpu.make_async_copy(
                        tab_hbm.at[pl.ds(0, D), pl.ds(0, CW)],
                        win.at[0], sem0).wait()

                @pl.when(p == 1)
                def _():
                    pltpu.make_async_copy(
                        tab_hbm.at[pl.ds(0, D), pl.ds(0, CW)],
                        win.at[1], sem1).wait()

                psplat = jnp.full((L,), p, jnp.int32)

                def grp_body(m, mc):
                    mr = mr_v[pl.ds(m * L, L)]
                    me = me_v[pl.ds(m * L, L)]
                    valid = iot < (cnt - m * L)
                    inch = (mr >= chunk_lo) & (mr < chunk_hi) & valid
                    ranks = plsc.cumsum(inch.astype(jnp.int32))
                    slots = (mc + ranks - 1) % RING

                    def ext_cond(carry):
                        mask, _ = carry
                        return plsc.all_reduce_population_count(mask)[0] > 0

                    def ext_body(carry):
                        mask, mcc = carry
                        k = plsc.all_reduce_ffs(mask)[0]
                        ksp = jnp.full((L,), k, jnp.int32)
                        r_k = jnp.take(mr, ksp)[0]
                        e_k = jnp.take(me, ksp)[0]
                        slot = jnp.take(slots, ksp)[0]
                        col = r_k - r0c
                        csp = jnp.full((L,), col, jnp.int32)
                        for j in range(D // L):
                            g = plsc.load_gather(
                                win, [psplat, iot + j * L, csp])
                            ring[pl.ds(slot * D + j * L, L)] = g

                        @pl.when(mcc >= RING)
                        def _():
                            pltpu.make_async_copy(
                                stag_hbm.at[0, pl.ds(0, D)],
                                ring.at[pl.ds(0, D)], ssem).wait()

                        pltpu.async_copy(
                            ring.at[pl.ds(slot * D, D)],
                            stag_hbm.at[e_k, pl.ds(colbase, D)], ssem)
                        return mask & (iot != k), mcc + 1

                    @pl.when(plsc.all_reduce_population_count(inch)[0] > 0)
                    def _():
                        lax.while_loop(ext_cond, ext_body, (inch, mc))

                    return mc + plsc.all_reduce_population_count(inch)[0]

                return lax.fori_loop(0, mg, grp_body, mc)

            return lax.fori_loop(0, nch, chunk_body, mc0)

        mc = run_pass(embt_hbm, mr0_v, me0_v, cnt_t, 0, 0)
        start_win(cembt_hbm, 0, 0)
        mc = run_pass(cembt_hbm, mr1_v, me1_v, cnt_c, D, mc)

        def fdrain(i, carry):
            @pl.when(i < jnp.minimum(mc, RING))
            def _():
                pltpu.make_async_copy(
                    stag_hbm.at[0, pl.ds(0, D)],
                    ring.at[pl.ds(0, D)], ssem).wait()
            return carry

        lax.fori_loop(0, RING, fdrain, 0)

    @functools.partial(
        pl.kernel,
        mesh=mesh,
        compiler_params=pltpu.CompilerParams(needs_layout_passes=False),
        out_type=jax.ShapeDtypeStruct((B,), jnp.float32),
        scratch_types=[
            pltpu.VMEM((BPW, 2 * D), jnp.float32),
            pltpu.VMEM((BPW,), jnp.float32),
            pltpu.SemaphoreType.DMA,
        ],
    )
    def dot_kernel(stag_hbm, out_hbm, rows_v, out_v, sem):
        wid = lax.axis_index("s") * NC + lax.axis_index("c")
        base = wid * BPW
        pltpu.async_copy(
            stag_hbm.at[pl.ds(base, BPW), pl.ds(0, 2 * D)], rows_v,
            sem).wait()
        lane = lax.iota(jnp.int32, L)

        def g_body(g, carry):
            res = jnp.zeros((L,), jnp.float32)
            for k in range(L):
                row = g * L + k
                acc = (rows_v[row, pl.ds(0, L)]
                       * rows_v[row, pl.ds(D, L)])
                for j in range(1, D // L):
                    acc = acc + (rows_v[row, pl.ds(j * L, L)]
                                 * rows_v[row, pl.ds(D + j * L, L)])
                res = jnp.where(lane == k, jnp.sum(acc), res)
            out_v[pl.ds(g * L, L)] = 1.0 / (1.0 + jnp.exp(-res))
            return carry

        lax.fori_loop(0, BPW // L, g_body, 0)
        pltpu.sync_copy(out_v, out_hbm.at[pl.ds(base, BPW)])

    staging = gather_kernel(target_nodes, context_nodes,
                            embeddings.T, context_embeddings.T)
    return dot_kernel(staging)


# speedup vs baseline: 1.0126x; 1.0126x over previous
import functools

import jax
import jax.numpy as jnp
from jax import lax
from jax.experimental import pallas as pl
from jax.experimental.pallas import tpu as pltpu
from jax.experimental.pallas import tpu_sc as plsc

NC = 2
NS = 16
L = 16
NW = NC * NS
CW = 512
MCAP = 3072
RING = 64


def kernel(target_nodes, context_nodes, embeddings, context_embeddings):
    B = target_nodes.shape[0]
    N, D = embeddings.shape
    NT = (N + 127) // 128
    RMAX0 = ((N + 127) // 128) * 128 - CW
    BPW = B // NW

    mesh = plsc.VectorSubcoreMesh(core_axis_name="c", subcore_axis_name="s")

    @functools.partial(
        pl.kernel,
        mesh=mesh,
        compiler_params=pltpu.CompilerParams(needs_layout_passes=False),
        out_type=jax.ShapeDtypeStruct((B, 2 * D), jnp.float32),
        scratch_types=[
            pltpu.VMEM((B,), jnp.int32),
            pltpu.VMEM((B,), jnp.int32),
            pltpu.VMEM((MCAP,), jnp.int32),
            pltpu.VMEM((MCAP,), jnp.int32),
            pltpu.VMEM((MCAP,), jnp.int32),
            pltpu.VMEM((MCAP,), jnp.int32),
            pltpu.VMEM((2, D, CW), jnp.float32),
            pltpu.VMEM((RING * D,), jnp.float32),
            pltpu.SemaphoreType.DMA,
            pltpu.SemaphoreType.DMA,
            pltpu.SemaphoreType.DMA,
        ],
    )
    def gather_kernel(tgt_hbm, ctx_hbm, embt_hbm, cembt_hbm, stag_hbm,
                      tval_v, cval_v, mr0_v, me0_v, mr1_v, me1_v, win, ring,
                      sem0, sem1, ssem):
        wid = lax.axis_index("s") * NC + lax.axis_index("c")
        t_lo = (wid * NT) // NW
        t_hi = ((wid + 1) * NT) // NW
        lo_abs = t_lo * 128
        hi_abs = jnp.minimum(t_hi * 128, N)
        nch = (hi_abs - lo_abs + CW - 1) // CW

        iot = lax.iota(jnp.int32, L)

        def start_win(tab_hbm, cc, p):
            r0c = pl.multiple_of(
                jnp.minimum(lo_abs + cc * CW, RMAX0), 128)
            for q in range(D // 8):
                src = tab_hbm.at[pl.ds(q * 8, 8), pl.ds(r0c, CW)]

                @pl.when(p == 0)
                def _():
                    pltpu.async_copy(src, win.at[0, pl.ds(q * 8, 8),
                                                 pl.ds(0, CW)], sem0)

                @pl.when(p == 1)
                def _():
                    pltpu.async_copy(src, win.at[1, pl.ds(q * 8, 8),
                                                 pl.ds(0, CW)], sem1)

        start_win(embt_hbm, 0, 0)

        pltpu.sync_copy(tgt_hbm, tval_v)
        pltpu.sync_copy(ctx_hbm, cval_v)

        def mk_scan(val_v, mr_v, me_v):
            def scan_body(g, cur):
                v = val_v[pl.ds(g * L, L)]
                pos = g * L + iot
                m = (v >= lo_abs) & (v < hi_abs)
                plsc.store_compressed(mr_v.at[pl.ds(cur, L)], v, mask=m)
                plsc.store_compressed(me_v.at[pl.ds(cur, L)], pos, mask=m)
                return cur + plsc.all_reduce_population_count(m)[0]
            return scan_body

        cnt_t = lax.fori_loop(0, B // L, mk_scan(tval_v, mr0_v, me0_v), 0)
        cnt_c = lax.fori_loop(0, B // L, mk_scan(cval_v, mr1_v, me1_v), 0)

        def run_pass(tab_hbm, mr_v, me_v, cnt, colbase, mc0):
            mg = (cnt + L - 1) // L

            def chunk_body(cc, mc):
                p = cc % 2
                chunk_lo = lo_abs + cc * CW
                chunk_hi = jnp.minimum(chunk_lo + CW, hi_abs)
                r0c = jnp.minimum(chunk_lo, RMAX0)

                @pl.when(cc + 1 < nch)
                def _():
                    start_win(tab_hbm, cc + 1, 1 - p)

                @pl.when(p == 0)
                def _():
                    pltpu.make_async_copy(
                        tab_hbm.at[pl.ds(0, D), pl.ds(0, CW)],
                        win.at[0], sem0).wait()

                @pl.when(p == 1)
                def _():
                    pltpu.make_async_copy(
                        tab_hbm.at[pl.ds(0, D), pl.ds(0, CW)],
                        win.at[1], sem1).wait()

                psplat = jnp.full((L,), p, jnp.int32)

                def grp_body(m, mc):
                    mr = mr_v[pl.ds(m * L, L)]
                    me = me_v[pl.ds(m * L, L)]
                    valid = iot < (cnt - m * L)
                    inch = (mr >= chunk_lo) & (mr < chunk_hi) & valid
                    ranks = plsc.cumsum(inch.astype(jnp.int32))
                    slots = (mc + ranks - 1) % RING

                    def ext_cond(carry):
                        mask, _ = carry
                        return plsc.all_reduce_population_count(mask)[0] > 0

                    def ext_body(carry):
                        mask, mcc = carry
                        k = plsc.all_reduce_ffs(mask)[0]
                        ksp = jnp.full((L,), k, jnp.int32)
                        r_k = jnp.take(mr, ksp)[0]
                        e_k = jnp.take(me, ksp)[0]
                        slot = jnp.take(slots, ksp)[0]
                        col = r_k - r0c
                        csp = jnp.full((L,), col, jnp.int32)
                        for j in range(D // L):
                            g = plsc.load_gather(
                                win, [psplat, iot + j * L, csp])
                            ring[pl.ds(slot * D + j * L, L)] = g

                        @pl.when(mcc >= RING)
                        def _():
                            pltpu.make_async_copy(
                                stag_hbm.at[0, pl.ds(0, D)],
                                ring.at[pl.ds(0, D)], ssem).wait()

                        pltpu.async_copy(
                            ring.at[pl.ds(slot * D, D)],
                            stag_hbm.at[e_k, pl.ds(colbase, D)], ssem)
                        return mask & (iot != k), mcc + 1

                    @pl.when(plsc.all_reduce_population_count(inch)[0] > 0)
                    def _():
                        lax.while_loop(ext_cond, ext_body, (inch, mc))

                    return mc + plsc.all_reduce_population_count(inch)[0]

                return lax.fori_loop(0, mg, grp_body, mc)

            return lax.fori_loop(0, nch, chunk_body, mc0)

        mc = run_pass(embt_hbm, mr0_v, me0_v, cnt_t, 0, 0)
        start_win(cembt_hbm, 0, 0)
        mc = run_pass(cembt_hbm, mr1_v, me1_v, cnt_c, D, mc)

        def fdrain(i, carry):
            @pl.when(i < jnp.minimum(mc, RING))
            def _():
                pltpu.make_async_copy(
                    stag_hbm.at[0, pl.ds(0, D)],
                    ring.at[pl.ds(0, D)], ssem).wait()
            return carry

        lax.fori_loop(0, RING, fdrain, 0)

    @functools.partial(
        pl.kernel,
        mesh=mesh,
        compiler_params=pltpu.CompilerParams(needs_layout_passes=False),
        out_type=jax.ShapeDtypeStruct((B,), jnp.float32),
        scratch_types=[
            pltpu.VMEM((BPW, 2 * D), jnp.float32),
            pltpu.VMEM((BPW,), jnp.float32),
            pltpu.SemaphoreType.DMA,
        ],
    )
    def dot_kernel(stag_hbm, out_hbm, rows_v, out_v, sem):
        wid = lax.axis_index("s") * NC + lax.axis_index("c")
        base = wid * BPW
        pltpu.async_copy(
            stag_hbm.at[pl.ds(base, BPW), pl.ds(0, 2 * D)], rows_v,
            sem).wait()
        lane = lax.iota(jnp.int32, L)

        def g_body(g, carry):
            res = jnp.zeros((L,), jnp.float32)
            for k in range(L):
                row = g * L + k
                acc = (rows_v[row, pl.ds(0, L)]
                       * rows_v[row, pl.ds(D, L)])
                for j in range(1, D // L):
                    acc = acc + (rows_v[row, pl.ds(j * L, L)]
                                 * rows_v[row, pl.ds(D + j * L, L)])
                res = jnp.where(lane == k, jnp.sum(acc), res)
            out_v[pl.ds(g * L, L)] = 1.0 / (1.0 + jnp.exp(-res))
            return carry

        lax.fori_loop(0, BPW // L, g_body, 0)
        pltpu.sync_copy(out_v, out_hbm.at[pl.ds(base, BPW)])

    staging = gather_kernel(target_nodes, context_nodes,
                            embeddings.T, context_embeddings.T)
    return dot_kernel(staging)
